# baseline (speedup 1.0000x reference)
import jax
import jax.numpy as jnp
from jax import lax
from jax.experimental import pallas as pl

_F = 208
_NFIELD = 26
_E = 16
_FE = _F * _E
_LB = 1024


def _prep_body(v_ref, fi_ref, emb_ref, sq_ref):
    v = v_ref[...]
    fi = fi_ref[...]
    rows = lax.broadcasted_iota(jnp.int32, (_NFIELD, _F), 0)
    onehot_t = (rows == fi).astype(jnp.float32)
    emb = lax.dot_general(onehot_t, v, (((0,), (0,)), ((), ())),
                          preferred_element_type=jnp.float32,
                          precision=lax.Precision.HIGHEST)
    sq_ref[:, 0:1] = jnp.sum(emb, axis=1, keepdims=True)
    sq_ref[:, 1:2] = jnp.sum(emb * emb, axis=1, keepdims=True)
    r_iota = lax.broadcasted_iota(jnp.int32, (_FE, _F), 0)
    f_iota = lax.broadcasted_iota(jnp.int32, (_FE, _F), 1)
    sel = (r_iota // _E == f_iota).astype(jnp.bfloat16)
    emb_hi = emb.astype(jnp.bfloat16)
    emb_lo = (emb - emb_hi.astype(jnp.float32)).astype(jnp.bfloat16)
    dn = (((1,), (0,)), ((), ()))
    emb_rep = lax.dot_general(sel, emb_hi, dn,
                              preferred_element_type=jnp.float32)
    emb_rep += lax.dot_general(sel, emb_lo, dn,
                               preferred_element_type=jnp.float32)
    re_iota = lax.broadcasted_iota(jnp.int32, (_FE, _E), 0)
    e_iota = lax.broadcasted_iota(jnp.int32, (_FE, _E), 1)
    pick = (re_iota % _E == e_iota).astype(jnp.float32)
    emb_ref[...] = jnp.sum(emb_rep * pick, axis=1, keepdims=True)


def _main_body(x_ref, w_ref, emb_ref, sq_ref, out_ref, y_ref):
    xt = x_ref[...]
    x3 = lax.broadcast_in_dim(xt, (_F, _E, _LB), (0, 2))
    xrep = x3.reshape(_FE, _LB)
    out_ref[...] = xrep * emb_ref[...]

    wcol = w_ref[...]
    scol = sq_ref[:, 0:1]
    qcol = sq_ref[:, 1:2]
    lin = jnp.sum(xt * wcol, axis=0, keepdims=True)
    t = jnp.sum(xt * scol, axis=0, keepdims=True)
    qq = jnp.sum(xt * xt * qcol, axis=0, keepdims=True)
    inter = 0.5 * (t * t - qq)
    y_ref[...] = jnp.concatenate([lin, inter], axis=0)


def kernel(inputs, w, V, field_index):
    B = inputs.shape[0]
    emb_flat, sq = pl.pallas_call(
        _prep_body,
        in_specs=[
            pl.BlockSpec((_NFIELD, _E), lambda: (0, 0)),
            pl.BlockSpec((1, _F), lambda: (0, 0)),
        ],
        out_specs=[
            pl.BlockSpec((_FE, 1), lambda: (0, 0)),
            pl.BlockSpec((_F, 2), lambda: (0, 0)),
        ],
        out_shape=[
            jax.ShapeDtypeStruct((_FE, 1), jnp.float32),
            jax.ShapeDtypeStruct((_F, 2), jnp.float32),
        ],
    )(V, field_index.reshape(1, _F))

    grid = B // _LB
    out_t, y_t = pl.pallas_call(
        _main_body,
        grid=(grid,),
        in_specs=[
            pl.BlockSpec((_F, _LB), lambda i: (0, i)),
            pl.BlockSpec((_F, 1), lambda i: (0, 0)),
            pl.BlockSpec((_FE, 1), lambda i: (0, 0)),
            pl.BlockSpec((_F, 2), lambda i: (0, 0)),
        ],
        out_specs=[
            pl.BlockSpec((_FE, _LB), lambda i: (0, i)),
            pl.BlockSpec((2, _LB), lambda i: (0, i)),
        ],
        out_shape=[
            jax.ShapeDtypeStruct((_FE, B), jnp.float32),
            jax.ShapeDtypeStruct((2, B), jnp.float32),
        ],
    )(inputs.T, w.reshape(_F, 1), emb_flat, sq)
    return (y_t.T, out_t.T.reshape(B, _F, _E))
